# baseline (speedup 1.0000x reference)
import functools
import math

import jax
import jax.numpy as jnp
from jax import lax
from jax.experimental import pallas as pl
from jax.experimental.pallas import tpu as pltpu
from jax.experimental.pallas import tpu_sc as plsc

EMBED = 64
SCALE = math.sqrt(EMBED)

NC = 2
NS = 16
NW = NC * NS
LANES = 16

A_BLK = 128
NBUF = 4
PREF = 2

PACK_COLS = 128


def _pack_table(table_t):
    v = table_t.shape[1]
    grid = pl.cdiv(v, PACK_COLS)

    def body(t_ref, o_ref):
        a = t_ref[...]
        even = (
            lax.broadcasted_iota(jnp.int32, (EMBED, PACK_COLS // 2), 1) * 2
        )
        o_ref[:, 0:EMBED] = jnp.take_along_axis(a, even, axis=1).T
        o_ref[:, EMBED : 2 * EMBED] = jnp.take_along_axis(a, even + 1, axis=1).T

    return pl.pallas_call(
        body,
        grid=(grid,),
        in_specs=[pl.BlockSpec((EMBED, PACK_COLS), lambda i: (0, i))],
        out_specs=pl.BlockSpec((PACK_COLS // 2, 2 * EMBED), lambda i: (i, 0)),
        out_shape=jax.ShapeDtypeStruct((v // 2, 2 * EMBED), jnp.float32),
    )(table_t)


def _build_lookup(b0, b1):
    assert b0 == NW * A_BLK
    n_chunk = b1
    assert n_chunk % NBUF == 0
    mesh = plsc.VectorSubcoreMesh(core_axis_name="c", subcore_axis_name="s")

    @functools.partial(
        pl.kernel,
        mesh=mesh,
        out_type=jax.ShapeDtypeStruct(
            (b1, EMBED // 8, NW, 8, A_BLK), jnp.float32
        ),
        compiler_params=pltpu.CompilerParams(
            use_tc_tiling_on_sc=False, needs_layout_passes=False
        ),
        scratch_types=(
            [pltpu.VMEM((n_chunk, A_BLK), jnp.int32)]
            + [pltpu.VMEM((NBUF, A_BLK), jnp.int32)]
            + [pltpu.VMEM((A_BLK, 2 * EMBED), jnp.float32) for _ in range(NBUF)]
            + [pltpu.VMEM((8, 8, A_BLK + 1), jnp.float32) for _ in range(NBUF)]
            + [pltpu.SemaphoreType.DMA for _ in range(2 * NBUF)]
        ),
    )
    def lookup(idx_hbm, table_hbm, out_hbm, idx_v, qbuf, *scratch):
        gbufs = scratch[:NBUF]
        tbufs = scratch[NBUF : 2 * NBUF]
        gsem = scratch[2 * NBUF : 3 * NBUF]
        ssem = scratch[3 * NBUF :]

        wid = lax.axis_index("s") * NC + lax.axis_index("c")
        a0 = wid * A_BLK
        pltpu.sync_copy(idx_hbm.at[:, pl.ds(a0, A_BLK)], idx_v)

        iota = lax.iota(jnp.int32, LANES)
        cvec = [iota + LANES * j for j in range(EMBED // LANES)]
        tr_idx = [lax.shift_right_logical(cv, 3) for cv in cvec]
        r_idx = [lax.bitwise_and(cv, 7) for cv in cvec]

        def gather_start(c, b):
            for j in range(A_BLK // LANES):
                sl = pl.ds(j * LANES, LANES)
                qbuf[b, sl] = lax.shift_right_logical(idx_v[c, sl], 1)
            pltpu.async_copy(table_hbm.at[qbuf.at[b]], gbufs[b], gsem[b])

        def gather_wait(c, b):
            pltpu.make_async_copy(
                table_hbm.at[qbuf.at[b]], gbufs[b], gsem[b]
            ).wait()

        def out_slice(c):
            return out_hbm.at[c, :, wid]

        def tbuf_slice(b):
            return tbufs[b].at[:, :, pl.ds(0, A_BLK)]

        def store_start(c, b):
            pltpu.async_copy(tbuf_slice(b), out_slice(c), ssem[b])

        def store_wait(c, b):
            pltpu.make_async_copy(tbuf_slice(b), out_slice(c), ssem[b]).wait()

        for c in range(PREF):
            gather_start(c, c)

        def outer(i, carry):
            for b in range(NBUF):
                c = i * NBUF + b
                gather_wait(c, b)

                gbuf, tbuf = gbufs[b], tbufs[b]
                crow = jnp.full((LANES,), c, jnp.int32)

                @plsc.parallel_loop(0, A_BLK, unroll=4)
                def _transpose_scale(i2):
                    col = jnp.full((LANES,), i2, jnp.int32)
                    pvec = plsc.load_gather(idx_v, [crow, col])
                    base = lax.bitwise_and(pvec, 1) * EMBED
                    for j in range(EMBED // LANES):
                        v = plsc.load_gather(gbuf, [col, base + cvec[j]])
                        plsc.store_scatter(
                            tbuf, [tr_idx[j], r_idx[j], col], v * SCALE
                        )

                store_start(c, b)

                bt = (b + PREF) % NBUF
                ct = c + PREF

                @pl.when(ct < n_chunk)
                def _prefetch():
                    @pl.when(c >= NBUF - PREF)
                    def _drain():
                        store_wait(ct - NBUF, bt)

                    gather_start(ct, bt)

            return carry

        lax.fori_loop(0, n_chunk // NBUF, outer, 0)

        for b in range(NBUF):
            store_wait(n_chunk - NBUF + b, b)

    return lookup


def kernel(inputs, table):
    b0, b1 = inputs.shape
    idx_t = jnp.transpose(inputs).astype(jnp.int32)
    packed = _pack_table(jnp.transpose(table))
    out5 = _build_lookup(b0, b1)(idx_t, packed)
    return out5.transpose(2, 4, 0, 1, 3).reshape(b0, b1, EMBED)


# speedup vs baseline: 7.7779x; 7.7779x over previous
import functools
import math

import jax
import jax.numpy as jnp
from jax import lax
from jax.experimental import pallas as pl
from jax.experimental.pallas import tpu as pltpu
from jax.experimental.pallas import tpu_sc as plsc

EMBED = 64
SCALE = math.sqrt(EMBED)

NC = 2
NS = 16
NW = NC * NS
LANES = 16

A_BLK = 128
NBUF = 4
PREF = 2

PACK_COLS = 4096


def _pack_table(table_t):
    v = table_t.shape[1]
    grid = pl.cdiv(v, PACK_COLS)

    def body(t_ref, o_ref):
        o_ref[:, 0:EMBED] = t_ref[...].T

    return pl.pallas_call(
        body,
        grid=(grid,),
        in_specs=[pl.BlockSpec((EMBED, PACK_COLS), lambda i: (0, i))],
        out_specs=pl.BlockSpec((PACK_COLS, 2 * EMBED), lambda i: (i, 0)),
        out_shape=jax.ShapeDtypeStruct((v, 2 * EMBED), jnp.float32),
    )(table_t)


def _build_lookup(b0, b1):
    assert b0 == NW * A_BLK
    n_chunk = b1
    assert n_chunk % NBUF == 0
    mesh = plsc.VectorSubcoreMesh(core_axis_name="c", subcore_axis_name="s")

    @functools.partial(
        pl.kernel,
        mesh=mesh,
        out_type=jax.ShapeDtypeStruct(
            (b1, EMBED // 8, NW, 8, A_BLK), jnp.float32
        ),
        compiler_params=pltpu.CompilerParams(
            use_tc_tiling_on_sc=False, needs_layout_passes=False
        ),
        scratch_types=(
            [pltpu.VMEM((n_chunk, A_BLK), jnp.int32)]
            + [pltpu.VMEM((A_BLK, 2 * EMBED), jnp.float32) for _ in range(NBUF)]
            + [pltpu.VMEM((8, 8, A_BLK + 1), jnp.float32) for _ in range(NBUF)]
            + [pltpu.SemaphoreType.DMA for _ in range(2 * NBUF)]
        ),
    )
    def lookup(idx_hbm, table_hbm, out_hbm, idx_v, *scratch):
        gbufs = scratch[:NBUF]
        tbufs = scratch[NBUF : 2 * NBUF]
        gsem = scratch[2 * NBUF : 3 * NBUF]
        ssem = scratch[3 * NBUF :]

        wid = lax.axis_index("s") * NC + lax.axis_index("c")
        a0 = wid * A_BLK
        pltpu.sync_copy(idx_hbm.at[:, pl.ds(a0, A_BLK)], idx_v)

        iota = lax.iota(jnp.int32, LANES)
        cvec = [iota + LANES * j for j in range(EMBED // LANES)]
        tr_idx = [lax.shift_right_logical(cv, 3) for cv in cvec]
        r_idx = [lax.bitwise_and(cv, 7) for cv in cvec]

        def gather_start(c, b):
            pltpu.async_copy(table_hbm.at[idx_v.at[c]], gbufs[b], gsem[b])

        def gather_wait(c, b):
            pltpu.make_async_copy(
                table_hbm.at[idx_v.at[c]], gbufs[b], gsem[b]
            ).wait()

        def out_slice(c):
            return out_hbm.at[c, :, wid]

        def tbuf_slice(b):
            return tbufs[b].at[:, :, pl.ds(0, A_BLK)]

        def store_start(c, b):
            pltpu.async_copy(tbuf_slice(b), out_slice(c), ssem[b])

        def store_wait(c, b):
            pltpu.make_async_copy(tbuf_slice(b), out_slice(c), ssem[b]).wait()

        for c in range(PREF):
            gather_start(c, c)

        def outer(i, carry):
            for b in range(NBUF):
                c = i * NBUF + b
                gather_wait(c, b)

                gbuf, tbuf = gbufs[b], tbufs[b]

                @plsc.parallel_loop(0, A_BLK, unroll=4)
                def _transpose_scale(i2):
                    col = jnp.full((LANES,), i2, jnp.int32)
                    for j in range(EMBED // LANES):
                        v = gbuf[i2, pl.ds(j * LANES, LANES)] * SCALE
                        plsc.store_scatter(
                            tbuf, [tr_idx[j], r_idx[j], col], v
                        )

                store_start(c, b)

                bt = (b + PREF) % NBUF
                ct = c + PREF

                @pl.when(ct < n_chunk)
                def _prefetch():
                    @pl.when(c >= NBUF - PREF)
                    def _drain():
                        store_wait(ct - NBUF, bt)

                    gather_start(ct, bt)

            return carry

        lax.fori_loop(0, n_chunk // NBUF, outer, 0)

        for b in range(NBUF):
            store_wait(n_chunk - NBUF + b, b)

    return lookup


def kernel(inputs, table):
    b0, b1 = inputs.shape
    idx_t = jnp.transpose(inputs).astype(jnp.int32)
    packed = _pack_table(jnp.transpose(table))
    out5 = _build_lookup(b0, b1)(idx_t, packed)
    return out5.transpose(2, 4, 0, 1, 3).reshape(b0, b1, EMBED)


# speedup vs baseline: 8.1316x; 1.0455x over previous
import functools
import math

import jax
import jax.numpy as jnp
from jax import lax
from jax.experimental import pallas as pl
from jax.experimental.pallas import tpu as pltpu
from jax.experimental.pallas import tpu_sc as plsc

EMBED = 64
SCALE = math.sqrt(EMBED)

NC = 2
NS = 16
NW = NC * NS
LANES = 16

A_BLK = 128
NBUF = 4
PREF = 3

PACK_COLS = 4096


def _pack_table(table_t):
    v = table_t.shape[1]
    grid = pl.cdiv(v, PACK_COLS)

    def body(t_ref, o_ref):
        o_ref[:, 0:EMBED] = t_ref[...].T

    return pl.pallas_call(
        body,
        grid=(grid,),
        in_specs=[pl.BlockSpec((EMBED, PACK_COLS), lambda i: (0, i))],
        out_specs=pl.BlockSpec((PACK_COLS, 2 * EMBED), lambda i: (i, 0)),
        out_shape=jax.ShapeDtypeStruct((v, 2 * EMBED), jnp.float32),
    )(table_t)


def _build_lookup(b0, b1):
    assert b0 == NW * A_BLK
    n_chunk = b1
    assert n_chunk % NBUF == 0
    mesh = plsc.VectorSubcoreMesh(core_axis_name="c", subcore_axis_name="s")

    @functools.partial(
        pl.kernel,
        mesh=mesh,
        out_type=jax.ShapeDtypeStruct(
            (b1, EMBED // 8, NW, 8, A_BLK), jnp.float32
        ),
        compiler_params=pltpu.CompilerParams(
            use_tc_tiling_on_sc=False, needs_layout_passes=False
        ),
        scratch_types=(
            [pltpu.VMEM((n_chunk, A_BLK), jnp.int32)]
            + [pltpu.VMEM((A_BLK, 2 * EMBED), jnp.float32) for _ in range(NBUF)]
            + [pltpu.VMEM((8, 8, A_BLK + 1), jnp.float32) for _ in range(NBUF)]
            + [pltpu.SemaphoreType.DMA for _ in range(2 * NBUF)]
        ),
    )
    def lookup(idx_hbm, table_hbm, out_hbm, idx_v, *scratch):
        gbufs = scratch[:NBUF]
        tbufs = scratch[NBUF : 2 * NBUF]
        gsem = scratch[2 * NBUF : 3 * NBUF]
        ssem = scratch[3 * NBUF :]

        wid = lax.axis_index("s") * NC + lax.axis_index("c")
        a0 = wid * A_BLK
        pltpu.sync_copy(idx_hbm.at[:, pl.ds(a0, A_BLK)], idx_v)

        iota = lax.iota(jnp.int32, LANES)
        cvec = [iota + LANES * j for j in range(EMBED // LANES)]
        tr_idx = [lax.shift_right_logical(cv, 3) for cv in cvec]
        r_idx = [lax.bitwise_and(cv, 7) for cv in cvec]

        def gather_start(c, b):
            pltpu.async_copy(table_hbm.at[idx_v.at[c]], gbufs[b], gsem[b])

        def gather_wait(c, b):
            pltpu.make_async_copy(
                table_hbm.at[idx_v.at[c]], gbufs[b], gsem[b]
            ).wait()

        def out_slice(c):
            return out_hbm.at[c, :, wid]

        def tbuf_slice(b):
            return tbufs[b].at[:, :, pl.ds(0, A_BLK)]

        def store_start(c, b):
            pltpu.async_copy(tbuf_slice(b), out_slice(c), ssem[b])

        def store_wait(c, b):
            pltpu.make_async_copy(tbuf_slice(b), out_slice(c), ssem[b]).wait()

        for c in range(PREF):
            gather_start(c, c)

        def outer(i, carry):
            for b in range(NBUF):
                c = i * NBUF + b
                gather_wait(c, b)

                gbuf, tbuf = gbufs[b], tbufs[b]

                @plsc.parallel_loop(0, A_BLK, unroll=4)
                def _transpose_scale(i2):
                    col = jnp.full((LANES,), i2, jnp.int32)
                    for j in range(EMBED // LANES):
                        v = gbuf[i2, pl.ds(j * LANES, LANES)] * SCALE
                        plsc.store_scatter(
                            tbuf, [tr_idx[j], r_idx[j], col], v
                        )

                store_start(c, b)

                bt = (b + PREF) % NBUF
                ct = c + PREF

                @pl.when(ct < n_chunk)
                def _prefetch():
                    @pl.when(c >= NBUF - PREF)
                    def _drain():
                        store_wait(ct - NBUF, bt)

                    gather_start(ct, bt)

            return carry

        lax.fori_loop(0, n_chunk // NBUF, outer, 0)

        for b in range(NBUF):
            store_wait(n_chunk - NBUF + b, b)

    return lookup


def kernel(inputs, table):
    b0, b1 = inputs.shape
    idx_t = jnp.transpose(inputs).astype(jnp.int32)
    packed = _pack_table(jnp.transpose(table))
    out5 = _build_lookup(b0, b1)(idx_t, packed)
    return out5.transpose(2, 4, 0, 1, 3).reshape(b0, b1, EMBED)
